# Initial kernel scaffold; baseline (speedup 1.0000x reference)
import jax
import jax.numpy as jnp
from jax.experimental import pallas as pl


def kernel(x, ln_c_g, ln_c_b, W_c, b_c, ln_s_g, ln_s_b, W_s, b_s, cb_c, cb_s, W_comb, b_comb, ln_o_g, ln_o_b):
    raise NotImplementedError("write your pallas kernel here")


# speedup vs baseline: 1.9118x; 1.9118x over previous
import functools

import jax
import jax.numpy as jnp
from jax import lax
from jax.experimental import pallas as pl
from jax.experimental.pallas import tpu as pltpu
from jax.experimental.pallas import tpu_sc as plsc

_HI = jax.lax.Precision.HIGHEST

_NTOK = 8192
_D = 1024
_HALF = 512
_K = 1024
_TILE = 512
_NB = _NTOK // _TILE

_NC = 2
_NS = 16
_NW = _NC * _NS
_RPW = _NTOK // _NW
_CHUNK = 32
_NCHUNK = _RPW // _CHUNK


def _prep_body(cb_c_ref, cb_s_ref, w_comb_ref,
               cbn_c_ref, cbn_s_ref, t_c_ref, t_s_ref):
    cbc = cb_c_ref[...]
    nc = jnp.sqrt(jnp.sum(cbc * cbc, axis=1, keepdims=True))
    cbn_c = cbc / jnp.maximum(nc, 1e-12)
    cbn_c_ref[...] = cbn_c.astype(jnp.bfloat16)
    cbs = cb_s_ref[...]
    ns = jnp.sqrt(jnp.sum(cbs * cbs, axis=1, keepdims=True))
    cbn_s = cbs / jnp.maximum(ns, 1e-12)
    cbn_s_ref[...] = cbn_s.astype(jnp.bfloat16)
    t_c_ref[...] = jnp.dot(cbn_c.astype(jnp.bfloat16),
                           w_comb_ref[0:_HALF, :].astype(jnp.bfloat16),
                           preferred_element_type=jnp.float32)
    t_s_ref[...] = jnp.dot(cbn_s.astype(jnp.bfloat16),
                           w_comb_ref[_HALF:, :].astype(jnp.bfloat16),
                           preferred_element_type=jnp.float32)


def _main_body(x_ref, lncg_ref, lncb_ref, wc_ref, bc_ref,
               lnsg_ref, lnsb_ref, ws_ref, bs_ref,
               cbnc_ref, cbns_ref,
               cidx_ref, sidx_ref, acc_ref):
    i = pl.program_id(0)
    x = x_ref[...]
    mu = jnp.mean(x, axis=1, keepdims=True)
    var = jnp.mean((x - mu) ** 2, axis=1, keepdims=True)
    xn = (x - mu) / jnp.sqrt(var + 1e-5)

    a_c = xn * lncg_ref[...] + lncb_ref[...]
    content = jnp.tanh(jnp.dot(a_c.astype(jnp.bfloat16),
                               wc_ref[...].astype(jnp.bfloat16),
                               preferred_element_type=jnp.float32)
                       + bc_ref[...])
    a_s = xn * lnsg_ref[...] + lnsb_ref[...]
    style = jnp.tanh(jnp.dot(a_s.astype(jnp.bfloat16),
                             ws_ref[...].astype(jnp.bfloat16),
                             preferred_element_type=jnp.float32)
                     + bs_ref[...])

    nc = jnp.sqrt(jnp.sum(content * content, axis=1, keepdims=True))
    cn = content / jnp.maximum(nc, 1e-12)
    ns = jnp.sqrt(jnp.sum(style * style, axis=1, keepdims=True))
    sn = style / jnp.maximum(ns, 1e-12)

    d_c = 1.0 - lax.dot_general(cn.astype(jnp.bfloat16), cbnc_ref[...],
                                (((1,), (1,)), ((), ())),
                                preferred_element_type=jnp.float32)
    d_s = 1.0 - lax.dot_general(sn.astype(jnp.bfloat16), cbns_ref[...],
                                (((1,), (1,)), ((), ())),
                                preferred_element_type=jnp.float32)
    ci = jnp.argmin(d_c, axis=1).astype(jnp.int32)
    si = jnp.argmin(d_s, axis=1).astype(jnp.int32)
    cidx_ref[0, 0, :] = ci
    sidx_ref[0, 0, :] = si

    dcmin = jnp.min(d_c, axis=1)
    dsmin = jnp.min(d_s, axis=1)
    ncf = nc[:, 0]
    nsf = ns[:, 0]
    e_c = jnp.sum(1.0 + ncf * ncf - 2.0 * ncf * (1.0 - dcmin))
    e_s = jnp.sum(1.0 + nsf * nsf - 2.0 * nsf * (1.0 - dsmin))
    cosv = jnp.sum(jnp.abs(jnp.sum(cn * sn, axis=1)))

    row = lax.broadcasted_iota(jnp.int32, (8, 128), 0)
    col = lax.broadcasted_iota(jnp.int32, (8, 128), 1)
    vals = (jnp.where((row == 0) & (col == 0), e_c, 0.0)
            + jnp.where((row == 1) & (col == 0), e_s, 0.0)
            + jnp.where((row == 2) & (col == 0), cosv, 0.0))

    @pl.when(i == 0)
    def _():
        acc_ref[...] = jnp.zeros((8, 128), jnp.float32)

    acc_ref[...] += vals


def _gather_body(tc_hbm, ts_hbm, cidx_hbm, sidx_hbm,
                 y1_hbm, y2_hbm,
                 ci_v, si_v, bufc, bufs, sem1, sem2):
    wid = lax.axis_index("s") * _NC + lax.axis_index("c")
    base = wid * _RPW
    pltpu.sync_copy(cidx_hbm.at[pl.ds(base, _RPW)], ci_v)
    pltpu.sync_copy(sidx_hbm.at[pl.ds(base, _RPW)], si_v)

    def body(k, carry):
        r0 = k * _CHUNK
        cpc = pltpu.async_copy(tc_hbm.at[ci_v.at[pl.ds(r0, _CHUNK)]], bufc,
                               sem1)
        cps = pltpu.async_copy(ts_hbm.at[si_v.at[pl.ds(r0, _CHUNK)]], bufs,
                               sem2)
        cpc.wait()
        pltpu.sync_copy(bufc, y1_hbm.at[pl.ds(base + r0, _CHUNK)])
        cps.wait()
        pltpu.sync_copy(bufs, y2_hbm.at[pl.ds(base + r0, _CHUNK)])
        return carry

    lax.fori_loop(0, _NCHUNK, body, 0)


def _ln_body(y1_ref, y2_ref, bcomb_ref, g_ref, b_ref, out_ref):
    y = y1_ref[...] + y2_ref[...] + bcomb_ref[...]
    mu = jnp.mean(y, axis=1, keepdims=True)
    var = jnp.mean((y - mu) ** 2, axis=1, keepdims=True)
    out_ref[...] = (y - mu) / jnp.sqrt(var + 1e-5) * g_ref[...] + b_ref[...]


def kernel(x, ln_c_g, ln_c_b, W_c, b_c, ln_s_g, ln_s_b, W_s, b_s,
           cb_c, cb_s, W_comb, b_comb, ln_o_g, ln_o_b):
    B, S, D = x.shape
    x2d = x.reshape(B * S, D)

    cbn_c, cbn_s, t_c, t_s = pl.pallas_call(
        _prep_body,
        out_shape=[
            jax.ShapeDtypeStruct((_K, _HALF), jnp.bfloat16),
            jax.ShapeDtypeStruct((_K, _HALF), jnp.bfloat16),
            jax.ShapeDtypeStruct((_K, _D), jnp.float32),
            jax.ShapeDtypeStruct((_K, _D), jnp.float32),
        ],
    )(cb_c, cb_s, W_comb)

    full = lambda shape: pl.BlockSpec(shape, lambda i: (0,) * len(shape))
    cidx3, sidx3, acc = pl.pallas_call(
        _main_body,
        grid=(_NB,),
        in_specs=[
            pl.BlockSpec((_TILE, _D), lambda i: (i, 0)),
            full((1, _D)), full((1, _D)),
            full((_D, _HALF)), full((1, _HALF)),
            full((1, _D)), full((1, _D)),
            full((_D, _HALF)), full((1, _HALF)),
            full((_K, _HALF)), full((_K, _HALF)),
        ],
        out_specs=[
            pl.BlockSpec((1, 1, _TILE), lambda i: (i, 0, 0)),
            pl.BlockSpec((1, 1, _TILE), lambda i: (i, 0, 0)),
            pl.BlockSpec((8, 128), lambda i: (0, 0)),
        ],
        out_shape=[
            jax.ShapeDtypeStruct((_NB, 1, _TILE), jnp.int32),
            jax.ShapeDtypeStruct((_NB, 1, _TILE), jnp.int32),
            jax.ShapeDtypeStruct((8, 128), jnp.float32),
        ],
    )(x2d,
      ln_c_g.reshape(1, _D), ln_c_b.reshape(1, _D), W_c, b_c.reshape(1, _HALF),
      ln_s_g.reshape(1, _D), ln_s_b.reshape(1, _D), W_s, b_s.reshape(1, _HALF),
      cbn_c, cbn_s)

    cidx_flat = cidx3.reshape(_NTOK)
    sidx_flat = sidx3.reshape(_NTOK)

    sc_gather = functools.partial(
        pl.kernel,
        mesh=plsc.VectorSubcoreMesh(core_axis_name="c", subcore_axis_name="s"),
        out_type=[
            jax.ShapeDtypeStruct((_NTOK, _D), jnp.float32),
            jax.ShapeDtypeStruct((_NTOK, _D), jnp.float32),
        ],
        scratch_types=[
            pltpu.VMEM((_RPW,), jnp.int32),
            pltpu.VMEM((_RPW,), jnp.int32),
            pltpu.VMEM((_CHUNK, _D), jnp.float32),
            pltpu.VMEM((_CHUNK, _D), jnp.float32),
            pltpu.SemaphoreType.DMA,
            pltpu.SemaphoreType.DMA,
        ],
    )
    y1, y2 = sc_gather(_gather_body)(t_c, t_s, cidx_flat, sidx_flat)

    combined2d = pl.pallas_call(
        _ln_body,
        grid=(_NB,),
        in_specs=[
            pl.BlockSpec((_TILE, _D), lambda i: (i, 0)),
            pl.BlockSpec((_TILE, _D), lambda i: (i, 0)),
            full((1, _D)), full((1, _D)), full((1, _D)),
        ],
        out_specs=pl.BlockSpec((_TILE, _D), lambda i: (i, 0)),
        out_shape=jax.ShapeDtypeStruct((_NTOK, _D), jnp.float32),
    )(y1, y2, b_comb.reshape(1, _D), ln_o_g.reshape(1, _D),
      ln_o_b.reshape(1, _D))

    e_c = acc[0, 0]
    e_s = acc[1, 0]
    cos_sum = acc[2, 0]
    closs = 0.1 * (e_c / (_NTOK * _HALF))
    sloss = 0.1 * (e_s / (_NTOK * _HALF))
    disentangle_loss = jnp.clip(cos_sum / _NTOK, 0.0, 1.0)
    total_loss = closs + sloss + 0.5 * disentangle_loss

    combined = combined2d.reshape(B, S, D)
    cidx = cidx_flat.reshape(B, S)
    sidx = sidx_flat.reshape(B, S)
    return combined, total_loss, cidx, sidx, disentangle_loss
